# baseline (speedup 1.0000x reference)
import functools

import jax
import jax.numpy as jnp
from jax import lax
from jax.experimental import pallas as pl
from jax.experimental.pallas import tpu as pltpu
from jax.experimental.pallas import tpu_sc as plsc

N_VOCAB = 1000
EMB = 64
B = 16384
KP1 = 11
KPAD = 16
GCOLS = 1024

_info = plsc.get_sparse_core_info()
_NC, _NS = _info.num_cores, _info.num_subcores
NW = _NC * _NS
BPW = B // NW
PAIRS = BPW * KPAD
IDX_ROWS = PAIRS // 128
NGRP = BPW // 16


def _mm_body(v_ref, u_ref, g_ref):
    g_ref[...] = lax.dot_general(
        v_ref[...], u_ref[...],
        dimension_numbers=(((1,), (1,)), ((), ())),
        preferred_element_type=jnp.float32)


def _softmax_body(x_ref, o_ref):
    x = x_ref[...]
    row = lax.broadcasted_iota(jnp.int32, x.shape, 0)
    xm = jnp.where(row < KP1, x, -1e30)
    m = jnp.max(xm, axis=0)
    e = jnp.exp(xm - m[None, :])
    s = jnp.sum(e, axis=0)
    per_row = m + jnp.log(s) - x[0, :]
    o_ref[0, 0] = jnp.sum(per_row) / B


_mesh = plsc.VectorSubcoreMesh(core_axis_name="c", subcore_axis_name="s")


@functools.partial(
    pl.kernel,
    mesh=_mesh,
    compiler_params=pltpu.CompilerParams(use_tc_tiling_on_sc=False),
    out_type=jax.ShapeDtypeStruct((KPAD * B, 1), jnp.float32),
    scratch_types=[
        pltpu.VMEM((BPW,), jnp.int32),
        pltpu.VMEM((KPAD * BPW,), jnp.int32),
        pltpu.VMEM((IDX_ROWS, 128), jnp.int32),
        pltpu.VMEM((PAIRS, 1), jnp.float32),
        pltpu.SemaphoreType.DMA,
    ],
)
def _gather_sc(cents_hbm, conts_hbm, g2_hbm, out_hbm,
               cents_v, conts_v, ridx, dest, sem_g):
    wid = lax.axis_index("s") * _NC + lax.axis_index("c")
    base = wid * BPW
    pltpu.sync_copy(cents_hbm.at[pl.ds(base, BPW)], cents_v)
    for k in range(KPAD):
        pltpu.sync_copy(conts_hbm.at[pl.ds(k * B + base, BPW)],
                        conts_v.at[pl.ds(k * BPW, BPW)])

    for g in range(NGRP):
        cvec = cents_v[pl.ds(g * 16, 16)] * GCOLS
        for k in range(KPAD):
            p = k * BPW + g * 16
            idx = cvec + conts_v[pl.ds(p, 16)]
            ridx[p // 128, pl.ds(p % 128, 16)] = idx

    copies = [
        pltpu.async_copy(g2_hbm.at[ridx.at[j]],
                         dest.at[pl.ds(j * 128, 128)], sem_g)
        for j in range(IDX_ROWS)
    ]
    for c in copies:
        c.wait()
    for k in range(KPAD):
        pltpu.sync_copy(dest.at[pl.ds(k * BPW, BPW)],
                        out_hbm.at[pl.ds(k * B + base, BPW)])


def kernel(cents, conts_negs, V, U):
    cents = cents.astype(jnp.int32)
    conts = conts_negs.astype(jnp.int32)
    conts_t = jnp.pad(conts, ((0, 0), (0, KPAD - KP1))).T
    u_pad = jnp.pad(U, ((0, GCOLS - N_VOCAB), (0, 0)))

    g = pl.pallas_call(
        _mm_body,
        out_shape=jax.ShapeDtypeStruct((N_VOCAB, GCOLS), jnp.float32),
    )(V, u_pad)

    flat = _gather_sc(cents, conts_t.reshape(-1),
                      g.reshape(N_VOCAB * GCOLS, 1))
    logits = flat.reshape(KPAD, B)

    out = pl.pallas_call(
        _softmax_body,
        out_shape=jax.ShapeDtypeStruct((1, 1), jnp.float32),
        out_specs=pl.BlockSpec(memory_space=pltpu.SMEM),
    )(logits)
    return out[0, 0]


# speedup vs baseline: 14.7898x; 14.7898x over previous
import functools

import jax
import jax.numpy as jnp
from jax import lax
from jax.experimental import pallas as pl
from jax.experimental.pallas import tpu as pltpu
from jax.experimental.pallas import tpu_sc as plsc

N_VOCAB = 1000
EMB = 64
B = 16384
KP1 = 11
NPAIR = KP1 * B
NROWS = NPAIR // 128
GBLK = 8

_info = plsc.get_sparse_core_info()
_NC, _NS = _info.num_cores, _info.num_subcores
NW = _NC * _NS
RPW = NROWS // NW


def _mm_body(v_ref, u_ref, g_ref):
    g_ref[...] = lax.dot_general(
        v_ref[...], u_ref[...],
        dimension_numbers=(((1,), (1,)), ((), ())),
        preferred_element_type=jnp.float32)


def _idx_body(c_ref, x_ref, o_ref):
    cm = c_ref[...].reshape(128, 128) * 128
    x = x_ref[...]
    rows = []
    for k in range(KP1):
        xk = x[:, k].reshape(128, 128)
        rows.append((xk >> 7) * 128000 + cm + (xk & 127))
    o_ref[...] = jnp.concatenate(rows, axis=0)


def _softmax_body(x_ref, o_ref):
    x = x_ref[...]
    blocks = [x[k * 128:(k + 1) * 128, :] for k in range(KP1)]
    m = blocks[0]
    for bl in blocks[1:]:
        m = jnp.maximum(m, bl)
    s = jnp.zeros_like(m)
    for bl in blocks:
        s = s + jnp.exp(bl - m)
    per = m + jnp.log(s) - blocks[0]
    o_ref[0, 0] = jnp.sum(per) / B


_mesh = plsc.VectorSubcoreMesh(core_axis_name="c", subcore_axis_name="s")


@functools.partial(
    pl.kernel,
    mesh=_mesh,
    compiler_params=pltpu.CompilerParams(use_tc_tiling_on_sc=False),
    out_type=jax.ShapeDtypeStruct((NROWS, 128), jnp.float32),
    scratch_types=[
        pltpu.VMEM((RPW, 128), jnp.int32),
        pltpu.VMEM((RPW, 128), jnp.float32),
        pltpu.SemaphoreType.DMA,
    ],
)
def _gather_sc(idx_hbm, g_hbm, out_hbm, ridx, dest, sem_g):
    wid = lax.axis_index("s") * _NC + lax.axis_index("c")
    base = wid * RPW
    pltpu.sync_copy(idx_hbm.at[pl.ds(base, RPW)], ridx)
    copies = [
        pltpu.async_copy(g_hbm.at[ridx.at[j]], dest.at[j], sem_g)
        for j in range(RPW)
    ]
    for c in copies:
        c.wait()
    pltpu.sync_copy(dest, out_hbm.at[pl.ds(base, RPW)])


def kernel(cents, conts_negs, V, U):
    cents = cents.astype(jnp.int32)
    conts = conts_negs.astype(jnp.int32)

    g = pl.pallas_call(
        _mm_body,
        grid=(GBLK,),
        in_specs=[
            pl.BlockSpec((N_VOCAB, EMB), lambda t: (0, 0)),
            pl.BlockSpec((128, EMB), lambda t: (t, 0)),
        ],
        out_specs=pl.BlockSpec((N_VOCAB, 128), lambda t: (t, 0)),
        out_shape=jax.ShapeDtypeStruct((GBLK * N_VOCAB, 128), jnp.float32),
    )(V, U)

    idx = pl.pallas_call(
        _idx_body,
        out_shape=jax.ShapeDtypeStruct((NROWS, 128), jnp.int32),
    )(cents, conts)

    logits = _gather_sc(idx, g.reshape(GBLK * N_VOCAB * 128))

    out = pl.pallas_call(
        _softmax_body,
        out_shape=jax.ShapeDtypeStruct((1, 1), jnp.float32),
        out_specs=pl.BlockSpec(memory_space=pltpu.SMEM),
    )(logits)
    return out[0, 0]


# speedup vs baseline: 21.1161x; 1.4277x over previous
import functools

import jax
import jax.numpy as jnp
from jax import lax
from jax.experimental import pallas as pl
from jax.experimental.pallas import tpu as pltpu
from jax.experimental.pallas import tpu_sc as plsc

N_VOCAB = 1000
EMB = 64
B = 16384
KP1 = 11
NPAIR = KP1 * B
NROWS = NPAIR // 128
GBLK = 8

_info = plsc.get_sparse_core_info()
_NC, _NS = _info.num_cores, _info.num_subcores
NW = _NC * _NS
RPW = NROWS // NW


def _mm_body(v_ref, u_ref, g_ref):
    g = lax.dot_general(
        v_ref[...], u_ref[...],
        dimension_numbers=(((1,), (1,)), ((), ())),
        preferred_element_type=jnp.float32)
    g_ref[...] = g.reshape(N_VOCAB * 128)


def _idx_body(c_ref, x_ref, o_ref):
    x = x_ref[...]
    c = c_ref[...]
    y = (x >> 7) * 128000 + c[:, None] * 128 + (x & 127)
    o_ref[...] = y.T.reshape(NROWS, 128)


def _softmax_body(x_ref, o_ref):
    x = x_ref[...]
    blocks = [x[k * 128:(k + 1) * 128, :] for k in range(KP1)]
    m = blocks[0]
    for bl in blocks[1:]:
        m = jnp.maximum(m, bl)
    s = jnp.zeros_like(m)
    for bl in blocks:
        s = s + jnp.exp(bl - m)
    per = m + jnp.log(s) - blocks[0]
    o_ref[0, 0] = jnp.sum(per) / B


_mesh = plsc.VectorSubcoreMesh(core_axis_name="c", subcore_axis_name="s")


@functools.partial(
    pl.kernel,
    mesh=_mesh,
    compiler_params=pltpu.CompilerParams(use_tc_tiling_on_sc=False),
    out_type=jax.ShapeDtypeStruct((NROWS, 128), jnp.float32),
    scratch_types=[
        pltpu.VMEM((RPW, 128), jnp.int32),
        pltpu.VMEM((RPW, 128), jnp.float32),
        pltpu.SemaphoreType.DMA,
    ],
)
def _gather_sc(idx_hbm, g_hbm, out_hbm, ridx, dest, sem_g):
    wid = lax.axis_index("s") * _NC + lax.axis_index("c")
    base = wid * RPW
    pltpu.sync_copy(idx_hbm.at[pl.ds(base, RPW)], ridx)
    copies = [
        pltpu.async_copy(g_hbm.at[ridx.at[j]], dest.at[j], sem_g)
        for j in range(RPW)
    ]
    for c in copies:
        c.wait()
    pltpu.sync_copy(dest, out_hbm.at[pl.ds(base, RPW)])


def kernel(cents, conts_negs, V, U):
    cents = cents.astype(jnp.int32)
    conts = conts_negs.astype(jnp.int32)

    g = pl.pallas_call(
        _mm_body,
        grid=(GBLK,),
        in_specs=[
            pl.BlockSpec((N_VOCAB, EMB), lambda t: (0, 0)),
            pl.BlockSpec((128, EMB), lambda t: (t, 0)),
        ],
        out_specs=pl.BlockSpec((N_VOCAB * 128,), lambda t: (t,)),
        out_shape=jax.ShapeDtypeStruct((GBLK * N_VOCAB * 128,), jnp.float32),
    )(V, U)

    idx = pl.pallas_call(
        _idx_body,
        out_shape=jax.ShapeDtypeStruct((NROWS, 128), jnp.int32),
    )(cents, conts)

    logits = _gather_sc(idx, g)

    out = pl.pallas_call(
        _softmax_body,
        out_shape=jax.ShapeDtypeStruct((1, 1), jnp.float32),
        out_specs=pl.BlockSpec(memory_space=pltpu.SMEM),
    )(logits)
    return out[0, 0]


# speedup vs baseline: 28.1766x; 1.3344x over previous
import functools

import jax
import jax.numpy as jnp
from jax import lax
from jax.experimental import pallas as pl
from jax.experimental.pallas import tpu as pltpu
from jax.experimental.pallas import tpu_sc as plsc

N_VOCAB = 1000
EMB = 64
B = 16384
KP1 = 11
NPAIR = KP1 * B
NROWS = NPAIR // 128
GBLK = 8

_info = plsc.get_sparse_core_info()
_NC, _NS = _info.num_cores, _info.num_subcores
NW = _NC * _NS
RPW = NROWS // NW


def _mm_body(v_ref, ut_ref, g_ref):
    g = lax.dot_general(
        v_ref[...], ut_ref[...],
        dimension_numbers=(((1,), (0,)), ((), ())),
        preferred_element_type=jnp.float32)
    g_ref[...] = g.reshape(N_VOCAB * 128)


def _idx_body(c_ref, xt_ref, o_ref):
    xt = xt_ref[...]
    c = c_ref[...]
    y = (xt >> 7) * 128000 + c[None, :] * 128 + (xt & 127)
    o_ref[...] = y.reshape(NROWS, 128)


def _softmax_body(x_ref, o_ref):
    x = x_ref[...]
    blocks = [x[k * 128:(k + 1) * 128, :] for k in range(KP1)]
    m = blocks[0]
    for bl in blocks[1:]:
        m = jnp.maximum(m, bl)
    s = jnp.zeros_like(m)
    for bl in blocks:
        s = s + jnp.exp(bl - m)
    per = m + jnp.log(s) - blocks[0]
    o_ref[0, 0] = jnp.sum(per) / B


_mesh = plsc.VectorSubcoreMesh(core_axis_name="c", subcore_axis_name="s")


@functools.partial(
    pl.kernel,
    mesh=_mesh,
    compiler_params=pltpu.CompilerParams(use_tc_tiling_on_sc=False),
    out_type=jax.ShapeDtypeStruct((NROWS, 128), jnp.float32),
    scratch_types=[
        pltpu.VMEM((RPW, 128), jnp.int32),
        pltpu.VMEM((RPW, 128), jnp.float32),
        pltpu.SemaphoreType.DMA,
    ],
)
def _gather_sc(idx_hbm, g_hbm, out_hbm, ridx, dest, sem_g):
    wid = lax.axis_index("s") * _NC + lax.axis_index("c")
    base = wid * RPW
    pltpu.sync_copy(idx_hbm.at[pl.ds(base, RPW)], ridx)
    copies = [
        pltpu.async_copy(g_hbm.at[ridx.at[j]], dest.at[j], sem_g)
        for j in range(RPW)
    ]
    for c in copies:
        c.wait()
    pltpu.sync_copy(dest, out_hbm.at[pl.ds(base, RPW)])


def kernel(cents, conts_negs, V, U):
    cents = cents.astype(jnp.int32)
    conts = conts_negs.astype(jnp.int32)

    g = pl.pallas_call(
        _mm_body,
        grid=(GBLK,),
        in_specs=[
            pl.BlockSpec((N_VOCAB, EMB), lambda t: (0, 0)),
            pl.BlockSpec((EMB, 128), lambda t: (0, t)),
        ],
        out_specs=pl.BlockSpec((N_VOCAB * 128,), lambda t: (t,)),
        out_shape=jax.ShapeDtypeStruct((GBLK * N_VOCAB * 128,), jnp.float32),
    )(V, U.T)

    idx = pl.pallas_call(
        _idx_body,
        out_shape=jax.ShapeDtypeStruct((NROWS, 128), jnp.int32),
    )(cents, conts.T)

    logits = _gather_sc(idx, g)

    out = pl.pallas_call(
        _softmax_body,
        out_shape=jax.ShapeDtypeStruct((1, 1), jnp.float32),
        out_specs=pl.BlockSpec(memory_space=pltpu.SMEM),
    )(logits)
    return out[0, 0]


# speedup vs baseline: 29.4747x; 1.0461x over previous
import functools

import jax
import jax.numpy as jnp
from jax import lax
from jax.experimental import pallas as pl
from jax.experimental.pallas import tpu as pltpu
from jax.experimental.pallas import tpu_sc as plsc

N_VOCAB = 1000
EMB = 64
B = 16384
KP1 = 11
NPAIR = KP1 * B
NROWS = NPAIR // 128
GBLK = 8

_info = plsc.get_sparse_core_info()
_NC, _NS = _info.num_cores, _info.num_subcores
NW = _NC * _NS
RPW = NROWS // NW


def _mm_body(vt_ref, ut_ref, g_ref):
    g = lax.dot_general(
        vt_ref[...], ut_ref[...],
        dimension_numbers=(((0,), (0,)), ((), ())),
        preferred_element_type=jnp.float32)
    g_ref[...] = g.reshape(N_VOCAB * 128)


def _idx_body(c_ref, xt_ref, o_ref):
    xt = xt_ref[...]
    c = c_ref[...]
    y = (xt >> 7) * 128000 + c[None, :] * 128 + (xt & 127)
    o_ref[...] = y.reshape(NROWS, 128)


def _softmax_body(x_ref, o_ref):
    x = x_ref[...]
    blocks = [x[k * 128:(k + 1) * 128, :] for k in range(KP1)]
    m = blocks[0]
    for bl in blocks[1:]:
        m = jnp.maximum(m, bl)
    s = jnp.zeros_like(m)
    for bl in blocks:
        s = s + jnp.exp(bl - m)
    per = m + jnp.log(s) - blocks[0]
    o_ref[0, 0] = jnp.sum(per) / B


_mesh = plsc.VectorSubcoreMesh(core_axis_name="c", subcore_axis_name="s")


@functools.partial(
    pl.kernel,
    mesh=_mesh,
    compiler_params=pltpu.CompilerParams(use_tc_tiling_on_sc=False),
    out_type=jax.ShapeDtypeStruct((NROWS, 128), jnp.float32),
    scratch_types=[
        pltpu.VMEM((RPW, 128), jnp.int32),
        pltpu.VMEM((RPW, 128), jnp.float32),
        pltpu.SemaphoreType.DMA,
    ],
)
def _gather_sc(idx_hbm, g_hbm, out_hbm, ridx, dest, sem_g):
    wid = lax.axis_index("s") * _NC + lax.axis_index("c")
    base = wid * RPW
    pltpu.sync_copy(idx_hbm.at[pl.ds(base, RPW)], ridx)
    copies = [
        pltpu.async_copy(g_hbm.at[ridx.at[j]], dest.at[j], sem_g)
        for j in range(RPW)
    ]
    for c in copies:
        c.wait()
    pltpu.sync_copy(dest, out_hbm.at[pl.ds(base, RPW)])


def kernel(cents, conts_negs, V, U):
    cents = cents.astype(jnp.int32)
    conts = conts_negs.astype(jnp.int32)

    g = pl.pallas_call(
        _mm_body,
        grid=(GBLK,),
        in_specs=[
            pl.BlockSpec((EMB, N_VOCAB), lambda t: (0, 0)),
            pl.BlockSpec((EMB, 128), lambda t: (0, t)),
        ],
        out_specs=pl.BlockSpec((N_VOCAB * 128,), lambda t: (t,)),
        out_shape=jax.ShapeDtypeStruct((GBLK * N_VOCAB * 128,), jnp.float32),
    )(V.T, U.T)

    idx = pl.pallas_call(
        _idx_body,
        out_shape=jax.ShapeDtypeStruct((NROWS, 128), jnp.int32),
    )(cents, conts.T)

    logits = _gather_sc(idx, g)

    out = pl.pallas_call(
        _softmax_body,
        out_shape=jax.ShapeDtypeStruct((1, 1), jnp.float32),
        out_specs=pl.BlockSpec(memory_space=pltpu.SMEM),
    )(logits)
    return out[0, 0]


# speedup vs baseline: 29.5961x; 1.0041x over previous
import functools

import jax
import jax.numpy as jnp
from jax import lax
from jax.experimental import pallas as pl
from jax.experimental.pallas import tpu as pltpu
from jax.experimental.pallas import tpu_sc as plsc

N_VOCAB = 1000
EMB = 64
B = 16384
KP1 = 11
NPAIR = KP1 * B
NROWS = NPAIR // 128
GBLK = 8

_info = plsc.get_sparse_core_info()
_NC, _NS = _info.num_cores, _info.num_subcores
NW = _NC * _NS
RPW = NROWS // NW


def _mm_body(vt_ref, ut_ref, g_ref):
    g = lax.dot_general(
        vt_ref[...], ut_ref[...],
        dimension_numbers=(((0,), (0,)), ((), ())),
        preferred_element_type=jnp.float32)
    g_ref[...] = g


def _idx_body(c_ref, xt_ref, o_ref):
    xt = xt_ref[...]
    c = c_ref[...]
    y = (xt >> 7) * 128000 + c[None, :] * 128 + (xt & 127)
    o_ref[...] = y.reshape(NROWS, 128)


def _softmax_body(x_ref, o_ref):
    x = x_ref[...]
    blocks = [x[k * 128:(k + 1) * 128, :] for k in range(KP1)]
    m = blocks[0]
    for bl in blocks[1:]:
        m = jnp.maximum(m, bl)
    s = jnp.zeros_like(m)
    for bl in blocks:
        s = s + jnp.exp(bl - m)
    per = m + jnp.log(s) - blocks[0]
    o_ref[0, 0] = jnp.sum(per) / B


_mesh = plsc.VectorSubcoreMesh(core_axis_name="c", subcore_axis_name="s")


@functools.partial(
    pl.kernel,
    mesh=_mesh,
    compiler_params=pltpu.CompilerParams(use_tc_tiling_on_sc=False),
    out_type=jax.ShapeDtypeStruct((NROWS, 128), jnp.float32),
    scratch_types=[
        pltpu.VMEM((RPW, 128), jnp.int32),
        pltpu.VMEM((RPW, 128), jnp.float32),
        pltpu.SemaphoreType.DMA,
    ],
)
def _gather_sc(idx_hbm, g_hbm, out_hbm, ridx, dest, sem_g):
    wid = lax.axis_index("s") * _NC + lax.axis_index("c")
    base = wid * RPW
    pltpu.sync_copy(idx_hbm.at[pl.ds(base, RPW)], ridx)
    copies = [
        pltpu.async_copy(g_hbm.at[ridx.at[j]], dest.at[j], sem_g)
        for j in range(RPW)
    ]
    for c in copies:
        c.wait()
    pltpu.sync_copy(dest, out_hbm.at[pl.ds(base, RPW)])


def kernel(cents, conts_negs, V, U):
    cents = cents.astype(jnp.int32)
    conts = conts_negs.astype(jnp.int32)

    g = pl.pallas_call(
        _mm_body,
        grid=(GBLK,),
        in_specs=[
            pl.BlockSpec((EMB, N_VOCAB), lambda t: (0, 0)),
            pl.BlockSpec((EMB, 128), lambda t: (0, t)),
        ],
        out_specs=pl.BlockSpec((N_VOCAB, 128), lambda t: (t, 0)),
        out_shape=jax.ShapeDtypeStruct((GBLK * N_VOCAB, 128), jnp.float32),
    )(V.T, U.T)

    idx = pl.pallas_call(
        _idx_body,
        out_shape=jax.ShapeDtypeStruct((NROWS, 128), jnp.int32),
    )(cents, conts.T)

    logits = _gather_sc(idx, g.reshape(GBLK * N_VOCAB * 128))

    out = pl.pallas_call(
        _softmax_body,
        out_shape=jax.ShapeDtypeStruct((1, 1), jnp.float32),
        out_specs=pl.BlockSpec(memory_space=pltpu.SMEM),
    )(logits)
    return out[0, 0]


# speedup vs baseline: 33.5444x; 1.1334x over previous
import functools

import jax
import jax.numpy as jnp
from jax import lax
from jax.experimental import pallas as pl
from jax.experimental.pallas import tpu as pltpu
from jax.experimental.pallas import tpu_sc as plsc

N_VOCAB = 1000
EMB = 64
B = 16384
KP1 = 11
NPAIR = KP1 * B
NROWS = NPAIR // 128
GBLK = 8

_info = plsc.get_sparse_core_info()
_NC, _NS = _info.num_cores, _info.num_subcores
NW = _NC * _NS
RPW = NROWS // NW


def _prep_body(vt_ref, ut_ref, c_ref, xt_ref, g_ref, idx_ref):
    vt = vt_ref[...]
    for t in range(GBLK):
        hi = min(N_VOCAB, (t + 1) * 128)
        gt = lax.dot_general(
            vt, ut_ref[:, t * 128:hi],
            dimension_numbers=(((0,), (0,)), ((), ())),
            preferred_element_type=jnp.float32)
        if hi - t * 128 < 128:
            gt = jnp.concatenate(
                [gt, jnp.zeros((N_VOCAB, 128 - (hi - t * 128)), jnp.float32)],
                axis=1)
        g_ref[t * N_VOCAB:(t + 1) * N_VOCAB, :] = gt
    xt = xt_ref[...]
    c = c_ref[...]
    y = (xt >> 7) * 128000 + c[None, :] * 128 + (xt & 127)
    idx_ref[...] = y.reshape(NROWS, 128)


def _softmax_body(x_ref, o_ref):
    x = x_ref[...]
    blocks = [x[k * 128:(k + 1) * 128, :] for k in range(KP1)]
    m = blocks[0]
    for bl in blocks[1:]:
        m = jnp.maximum(m, bl)
    s = jnp.zeros_like(m)
    for bl in blocks:
        s = s + jnp.exp(bl - m)
    per = m + jnp.log(s) - blocks[0]
    o_ref[0, 0] = jnp.sum(per) / B


_mesh = plsc.VectorSubcoreMesh(core_axis_name="c", subcore_axis_name="s")


@functools.partial(
    pl.kernel,
    mesh=_mesh,
    compiler_params=pltpu.CompilerParams(use_tc_tiling_on_sc=False),
    out_type=jax.ShapeDtypeStruct((NROWS, 128), jnp.float32),
    scratch_types=[
        pltpu.VMEM((RPW, 128), jnp.int32),
        pltpu.VMEM((RPW, 128), jnp.float32),
        pltpu.SemaphoreType.DMA,
    ],
)
def _gather_sc(idx_hbm, g_hbm, out_hbm, ridx, dest, sem_g):
    wid = lax.axis_index("s") * _NC + lax.axis_index("c")
    base = wid * RPW
    pltpu.sync_copy(idx_hbm.at[pl.ds(base, RPW)], ridx)
    copies = [
        pltpu.async_copy(g_hbm.at[ridx.at[j]], dest.at[j], sem_g)
        for j in range(RPW)
    ]
    for c in copies:
        c.wait()
    pltpu.sync_copy(dest, out_hbm.at[pl.ds(base, RPW)])


def kernel(cents, conts_negs, V, U):
    cents = cents.astype(jnp.int32)
    conts = conts_negs.astype(jnp.int32)

    g, idx = pl.pallas_call(
        _prep_body,
        out_shape=(
            jax.ShapeDtypeStruct((GBLK * N_VOCAB, 128), jnp.float32),
            jax.ShapeDtypeStruct((NROWS, 128), jnp.int32),
        ),
    )(V.T, U.T, cents, conts.T)

    logits = _gather_sc(idx, g.reshape(GBLK * N_VOCAB * 128))

    out = pl.pallas_call(
        _softmax_body,
        out_shape=jax.ShapeDtypeStruct((1, 1), jnp.float32),
        out_specs=pl.BlockSpec(memory_space=pltpu.SMEM),
    )(logits)
    return out[0, 0]
